# Initial kernel scaffold; baseline (speedup 1.0000x reference)
import jax
import jax.numpy as jnp
from jax.experimental import pallas as pl


def kernel(x, params, km5_in, km5_out, km3_0_in, km3_0_out, km3_1_in, km3_1_out, km3_2_in, km3_2_out, km3_3_in, km3_3_out, km3_4_in, km3_4_out, kmd01_in, kmd01_out, kmd12_in, kmd12_out, kmd23_in, kmd23_out, kmd34_in, kmd34_out):
    raise NotImplementedError("write your pallas kernel here")


# speedup vs baseline: 1.6430x; 1.6430x over previous
import functools

import jax
import jax.numpy as jnp
from jax import lax
from jax.experimental import pallas as pl
from jax.experimental.pallas import tpu as pltpu
from jax.experimental.pallas import tpu_sc as plsc

NC, NS = 2, 16
NW = NC * NS
CHUNK = 128
EDGE_ALIGN = NW * CHUNK
_EPS = 1e-5


def _rup(x, m):
    return (x + m - 1) // m * m


def _mesh():
    return plsc.VectorSubcoreMesh(core_axis_name="c", subcore_axis_name="s")


_SC_PARAMS = pltpu.CompilerParams(use_tc_tiling_on_sc=False)


@functools.cache
def _gather_kernel(n_in, D, E):
    epw = E // NW
    nch = epw // CHUNK

    @functools.partial(
        pl.kernel,
        out_type=jax.ShapeDtypeStruct((E, D), jnp.float32),
        mesh=_mesh(),
        compiler_params=_SC_PARAMS,
        scratch_types=[
            pltpu.VMEM((nch, CHUNK), jnp.int32),
            pltpu.VMEM((CHUNK, D), jnp.float32),
            pltpu.SemaphoreType.DMA,
        ],
    )
    def k(f_hbm, idx_hbm, g_hbm, idx_v, rows_v, sem):
        wid = lax.axis_index("s") * NC + lax.axis_index("c")
        pltpu.sync_copy(idx_hbm.at[wid], idx_v)

        def step(j, carry):
            pltpu.async_copy(f_hbm.at[idx_v.at[j]], rows_v, sem).wait()
            pltpu.sync_copy(rows_v, g_hbm.at[pl.ds(wid * epw + j * CHUNK, CHUNK)])
            return carry

        lax.fori_loop(0, nch, step, 0)

    return k


@functools.cache
def _scatter_kernel(E, D, R):
    D2 = D // 2
    eps = E // NS
    nch = eps // CHUNK
    rps = R // NS

    @functools.partial(
        pl.kernel,
        out_type=jax.ShapeDtypeStruct((R, D), jnp.float32),
        mesh=_mesh(),
        compiler_params=_SC_PARAMS,
        scratch_types=[
            pltpu.VMEM((nch, CHUNK), jnp.int32),
            pltpu.VMEM((CHUNK, D2), jnp.float32),
            pltpu.VMEM_SHARED((R, D2), jnp.float32),
            pltpu.SemaphoreType.DMA,
        ],
    )
    def k(m_hbm, oidx_hbm, z_hbm, out_hbm, idx_v, rows_v, acc, sem):
        cid = lax.axis_index("c")
        sid = lax.axis_index("s")
        col0 = cid * D2
        pltpu.sync_copy(z_hbm.at[pl.ds(sid * rps, rps)],
                        acc.at[pl.ds(sid * rps, rps)])
        plsc.subcore_barrier()
        pltpu.sync_copy(oidx_hbm.at[sid], idx_v)

        def step(j, carry):
            pltpu.sync_copy(
                m_hbm.at[pl.ds(sid * eps + j * CHUNK, CHUNK), pl.ds(col0, D2)],
                rows_v)
            pltpu.sync_copy(rows_v, acc.at[idx_v.at[j]], add=True)
            return carry

        lax.fori_loop(0, nch, step, 0)
        plsc.subcore_barrier()
        pltpu.sync_copy(acc.at[pl.ds(sid * rps, rps)],
                        out_hbm.at[pl.ds(sid * rps, rps), pl.ds(col0, D2)])

    return k


def _pad_map(idx, Mp, E, fill, workers):
    K, M = idx.shape
    a = jnp.full((K, Mp), fill, jnp.int32).at[:, :M].set(idx)
    flat = jnp.full((E,), fill, jnp.int32).at[: K * Mp].set(a.reshape(-1))
    return flat.reshape(workers, E // (workers * CHUNK), CHUNK)


def _offset_matmul(g, W, Mp, E):
    K, cin, cout = W.shape

    def body(g_ref, w_ref, o_ref):
        o_ref[...] = jnp.dot(g_ref[...], w_ref[0],
                             preferred_element_type=jnp.float32)

    return pl.pallas_call(
        body,
        grid=(K,),
        in_specs=[
            pl.BlockSpec((Mp, cin), lambda k: (k, 0)),
            pl.BlockSpec((1, cin, cout), lambda k: (k, 0, 0)),
        ],
        out_specs=pl.BlockSpec((Mp, cout), lambda k: (k, 0)),
        out_shape=jax.ShapeDtypeStruct((E, cout), jnp.float32),
    )(g, W)


def _row_block(n):
    if n <= 1024:
        return _rup(n, 8)
    return 1024


def _bn_stats(src, n):
    c = src.shape[1]
    Rb = _row_block(n)
    grid = -(-n // Rb)

    def body(p_ref, st_ref):
        i = pl.program_id(0)
        s = p_ref[...]
        rows = lax.broadcasted_iota(jnp.int32, (Rb, c), 0) + i * Rb
        s = jnp.where(rows < n, s, 0.0)
        blk = jnp.concatenate(
            [jnp.sum(s, axis=0, keepdims=True),
             jnp.sum(s * s, axis=0, keepdims=True),
             jnp.zeros((6, c), jnp.float32)], axis=0)

        @pl.when(i == 0)
        def _():
            st_ref[...] = jnp.zeros_like(st_ref)

        st_ref[...] += blk

    return pl.pallas_call(
        body,
        grid=(grid,),
        in_specs=[pl.BlockSpec((Rb, c), lambda i: (i, 0))],
        out_specs=pl.BlockSpec((8, c), lambda i: (0, 0)),
        out_shape=jax.ShapeDtypeStruct((8, c), jnp.float32),
    )(src)


def _bn_apply(src, st, n, gamma, beta, res, relu):
    c = src.shape[-1]
    Rb = _row_block(n)
    grid = -(-n // Rb)
    inv_n = 1.0 / n

    def body(*refs):
        if res is not None:
            p_ref, st_ref, g_ref, b_ref, r_ref, o_ref = refs
        else:
            p_ref, st_ref, g_ref, b_ref, o_ref = refs
        s = p_ref[...]
        mu = st_ref[0:1] * inv_n
        var = st_ref[1:2] * inv_n - mu * mu
        scale = lax.rsqrt(var + _EPS) * g_ref[...]
        y = (s - mu) * scale + b_ref[...]
        if res is not None:
            y = y + r_ref[...]
        if relu:
            y = jnp.maximum(y, 0.0)
        o_ref[...] = y

    in_specs = [
        pl.BlockSpec((Rb, c), lambda i: (i, 0)),
        pl.BlockSpec((8, c), lambda i: (0, 0)),
        pl.BlockSpec((1, c), lambda i: (0, 0)),
        pl.BlockSpec((1, c), lambda i: (0, 0)),
    ]
    args = [src, st, gamma.reshape(1, c), beta.reshape(1, c)]
    if res is not None:
        in_specs.append(pl.BlockSpec((Rb, c), lambda i: (i, 0)))
        args.append(res)

    return pl.pallas_call(
        body,
        grid=(grid,),
        in_specs=in_specs,
        out_specs=pl.BlockSpec((Rb, c), lambda i: (i, 0)),
        out_shape=jax.ShapeDtypeStruct((n, c), jnp.float32),
    )(*args)


def _matmul_stats(x, W):
    n, cin = x.shape
    cout = W.shape[1]
    Rb = _row_block(n)
    grid = -(-n // Rb)

    def body(x_ref, w_ref, o_ref, st_ref):
        i = pl.program_id(0)
        xw = jnp.dot(x_ref[...], w_ref[...], preferred_element_type=jnp.float32)
        rows = lax.broadcasted_iota(jnp.int32, (Rb, cout), 0) + i * Rb
        xs = jnp.where(rows < n, xw, 0.0)
        blk = jnp.concatenate(
            [jnp.sum(xs, axis=0, keepdims=True),
             jnp.sum(xs * xs, axis=0, keepdims=True),
             jnp.zeros((6, cout), jnp.float32)], axis=0)

        @pl.when(i == 0)
        def _():
            st_ref[...] = jnp.zeros_like(st_ref)

        st_ref[...] += blk
        o_ref[...] = xw

    return pl.pallas_call(
        body,
        grid=(grid,),
        in_specs=[
            pl.BlockSpec((Rb, cin), lambda i: (i, 0)),
            pl.BlockSpec((cin, cout), lambda i: (0, 0)),
        ],
        out_specs=[
            pl.BlockSpec((Rb, cout), lambda i: (i, 0)),
            pl.BlockSpec((8, cout), lambda i: (0, 0)),
        ],
        out_shape=[
            jax.ShapeDtypeStruct((n, cout), jnp.float32),
            jax.ShapeDtypeStruct((8, cout), jnp.float32),
        ],
    )(x, W)


def _final_matmul(x, W, b):
    n, cin = x.shape
    cout = W.shape[1]
    Rb = _row_block(n)
    grid = -(-n // Rb)

    def body(x_ref, w_ref, b_ref, o_ref):
        o_ref[...] = jnp.dot(x_ref[...], w_ref[...],
                             preferred_element_type=jnp.float32) + b_ref[...]

    return pl.pallas_call(
        body,
        grid=(grid,),
        in_specs=[
            pl.BlockSpec((Rb, cin), lambda i: (i, 0)),
            pl.BlockSpec((cin, cout), lambda i: (0, 0)),
            pl.BlockSpec((1, cout), lambda i: (0, 0)),
        ],
        out_specs=pl.BlockSpec((Rb, cout), lambda i: (i, 0)),
        out_shape=jax.ShapeDtypeStruct((n, cout), jnp.float32),
    )(x, W, b.reshape(1, cout))


def _sparse_conv(f, W, iidx, oidx, n_out):
    K, M = iidx.shape
    n_in, cin = f.shape
    cout = W.shape[2]
    if cin % 16:
        pad = 16 - cin % 16
        f = jnp.pad(f, ((0, 0), (0, pad)))
        W = jnp.pad(W, ((0, 0), (0, pad), (0, 0)))
        cin += pad
    Mp = _rup(M, 8)
    E = _rup(K * Mp, EDGE_ALIGN)
    R = _rup(n_out + 1, 128)
    iflat = _pad_map(iidx, Mp, E, 0, NW)
    oflat = _pad_map(oidx, Mp, E, n_out, NS)
    g = _gather_kernel(n_in, cin, E)(f, iflat)
    m = _offset_matmul(g, W, Mp, E)
    zeros = jnp.zeros((R, cout // 2), jnp.float32)
    return _scatter_kernel(E, cout, R)(m, oflat, zeros)


def _sconv_bn(f, W, iidx, oidx, n_out, gamma, beta, res=None, relu=True):
    s = _sparse_conv(f, W, iidx, oidx, n_out)
    st = _bn_stats(s, n_out)
    return _bn_apply(s, st, n_out, gamma, beta, res, relu)


def _matmul_bn(x, W, gamma, beta):
    xw, st = _matmul_stats(x, W)
    return _bn_apply(xw, st, x.shape[0], gamma, beta, None, False)


def _res_block(xf, p, iidx, oidx, n):
    h = _sconv_bn(xf, p["conv1"], iidx, oidx, n, p["bn1g"], p["bn1b"])
    if "down_w" in p:
        res = _matmul_bn(xf, p["down_w"], p["down_g"], p["down_b"])
    else:
        res = xf
    return _sconv_bn(h, p["conv2"], iidx, oidx, n, p["bn2g"], p["bn2b"],
                     res=res, relu=True)


def _res_layer(xf, ps, iidx, oidx, n):
    for p in ps:
        xf = _res_block(xf, p, iidx, oidx, n)
    return xf


def kernel(x, params, km5_in, km5_out, km3_0_in, km3_0_out, km3_1_in,
           km3_1_out, km3_2_in, km3_2_out, km3_3_in, km3_3_out, km3_4_in,
           km3_4_out, kmd01_in, kmd01_out, kmd12_in, kmd12_out, kmd23_in,
           kmd23_out, kmd34_in, kmd34_out):
    P = params
    n0, n1, n2, n3, n4 = 20000, 5000, 1250, 312, 78

    out_p1 = _sconv_bn(x, P["conv0"], km5_in, km5_out, n0,
                       P["bn0g"], P["bn0b"])
    out = _sconv_bn(out_p1, P["conv1"], kmd01_in, kmd01_out, n1,
                    P["bn1g"], P["bn1b"])
    out_b1 = _res_layer(out, P["block1"], km3_1_in, km3_1_out, n1)
    out = _sconv_bn(out_b1, P["conv2"], kmd12_in, kmd12_out, n2,
                    P["bn2g"], P["bn2b"])
    out_b2 = _res_layer(out, P["block2"], km3_2_in, km3_2_out, n2)
    out = _sconv_bn(out_b2, P["conv3"], kmd23_in, kmd23_out, n3,
                    P["bn3g"], P["bn3b"])
    out_b3 = _res_layer(out, P["block3"], km3_3_in, km3_3_out, n3)
    out = _sconv_bn(out_b3, P["conv4"], kmd34_in, kmd34_out, n4,
                    P["bn4g"], P["bn4b"])
    out = _res_layer(out, P["block4"], km3_4_in, km3_4_out, n4)
    out = _sconv_bn(out, P["convtr4"], kmd34_out, kmd34_in, n3,
                    P["bntr4g"], P["bntr4b"])
    out = jnp.concatenate([out, out_b3], axis=1)
    out = _res_layer(out, P["block5"], km3_3_in, km3_3_out, n3)
    out = _sconv_bn(out, P["convtr5"], kmd23_out, kmd23_in, n2,
                    P["bntr5g"], P["bntr5b"])
    out = jnp.concatenate([out, out_b2], axis=1)
    out = _res_layer(out, P["block6"], km3_2_in, km3_2_out, n2)
    out = _sconv_bn(out, P["convtr6"], kmd12_out, kmd12_in, n1,
                    P["bntr6g"], P["bntr6b"])
    out = jnp.concatenate([out, out_b1], axis=1)
    out = _res_layer(out, P["block7"], km3_1_in, km3_1_out, n1)
    out = _sconv_bn(out, P["convtr7"], kmd01_out, kmd01_in, n0,
                    P["bntr7g"], P["bntr7b"])
    out = jnp.concatenate([out, out_p1], axis=1)
    out = _res_layer(out, P["block8"], km3_0_in, km3_0_out, n0)
    return _final_matmul(out, P["final_w"], P["final_b"])


# speedup vs baseline: 1.7627x; 1.0729x over previous
import functools

import jax
import jax.numpy as jnp
from jax import lax
from jax.experimental import pallas as pl
from jax.experimental.pallas import tpu as pltpu
from jax.experimental.pallas import tpu_sc as plsc

NC, NS = 2, 16
NW = NC * NS
CHUNK = 128
EDGE_ALIGN = NW * CHUNK
_EPS = 1e-5


def _rup(x, m):
    return (x + m - 1) // m * m


def _mesh():
    return plsc.VectorSubcoreMesh(core_axis_name="c", subcore_axis_name="s")


_SC_PARAMS = pltpu.CompilerParams(use_tc_tiling_on_sc=False)


def _nbuf(d):
    return 4 if d <= 128 else 2


@functools.cache
def _gather_kernel(n_in, D, E):
    epw = E // NW
    nch = epw // CHUNK
    NB = _nbuf(D)
    ngroups = -(-nch // NB)

    @functools.partial(
        pl.kernel,
        out_type=jax.ShapeDtypeStruct((E, D), jnp.float32),
        mesh=_mesh(),
        compiler_params=_SC_PARAMS,
        scratch_types=(
            [pltpu.VMEM((nch, CHUNK), jnp.int32)]
            + [pltpu.VMEM((CHUNK, D), jnp.float32) for _ in range(NB)]
            + [pltpu.SemaphoreType.DMA for _ in range(2 * NB)]
        ),
    )
    def k(f_hbm, idx_hbm, g_hbm, idx_v, *bufs_sems):
        rows = bufs_sems[:NB]
        sg = bufs_sems[NB:2 * NB]
        sw = bufs_sems[2 * NB:]
        wid = lax.axis_index("s") * NC + lax.axis_index("c")
        pltpu.sync_copy(idx_hbm.at[wid], idx_v)
        for b in range(NB):
            if b < nch:
                pltpu.async_copy(f_hbm.at[idx_v.at[b]], rows[b], sg[b])

        def grp(t, carry):
            for b in range(NB):
                j = t * NB + b

                @pl.when(j < nch)
                def _(b=b, j=j):
                    pltpu.make_async_copy(
                        f_hbm.at[pl.ds(0, CHUNK)], rows[b], sg[b]).wait()
                    pltpu.async_copy(
                        rows[b],
                        g_hbm.at[pl.ds(wid * epw + j * CHUNK, CHUNK)], sw[b])
            for b in range(NB):
                j2 = (t + 1) * NB + b

                @pl.when(j2 < nch)
                def _(b=b, j2=j2):
                    pltpu.make_async_copy(
                        rows[b], g_hbm.at[pl.ds(0, CHUNK)], sw[b]).wait()
                    pltpu.async_copy(f_hbm.at[idx_v.at[j2]], rows[b], sg[b])
            return carry

        lax.fori_loop(0, ngroups, grp, 0)
        for b in range(NB):
            if b < nch:
                pltpu.make_async_copy(
                    rows[b], g_hbm.at[pl.ds(0, CHUNK)], sw[b]).wait()

    return k


@functools.cache
def _scatter_kernel(E, D, R):
    D2 = D // 2
    eps = E // NS
    nch = eps // CHUNK
    rps = R // NS
    NB = _nbuf(D2)
    ngroups = -(-nch // NB)

    @functools.partial(
        pl.kernel,
        out_type=jax.ShapeDtypeStruct((R, D), jnp.float32),
        mesh=_mesh(),
        compiler_params=_SC_PARAMS,
        scratch_types=(
            [pltpu.VMEM((nch, CHUNK), jnp.int32),
             pltpu.VMEM_SHARED((R, D2), jnp.float32)]
            + [pltpu.VMEM((CHUNK, D2), jnp.float32) for _ in range(NB)]
            + [pltpu.SemaphoreType.DMA for _ in range(NB)]
        ),
    )
    def k(m_hbm, oidx_hbm, z_hbm, out_hbm, idx_v, acc, *bufs_sems):
        rows = bufs_sems[:NB]
        sr = bufs_sems[NB:]
        cid = lax.axis_index("c")
        sid = lax.axis_index("s")
        col0 = cid * D2
        pltpu.sync_copy(z_hbm.at[pl.ds(sid * rps, rps)],
                        acc.at[pl.ds(sid * rps, rps)])
        plsc.subcore_barrier()
        pltpu.sync_copy(oidx_hbm.at[sid], idx_v)
        for b in range(NB):
            if b < nch:
                pltpu.async_copy(
                    m_hbm.at[pl.ds(sid * eps + b * CHUNK, CHUNK),
                             pl.ds(col0, D2)], rows[b], sr[b])

        def grp(t, carry):
            for b in range(NB):
                j = t * NB + b

                @pl.when(j < nch)
                def _(b=b, j=j):
                    pltpu.make_async_copy(
                        m_hbm.at[pl.ds(0, CHUNK), pl.ds(0, D2)],
                        rows[b], sr[b]).wait()
                    pltpu.sync_copy(rows[b],
                                    acc.at[idx_v.at[j]],
                                    add=True)
            for b in range(NB):
                j2 = (t + 1) * NB + b

                @pl.when(j2 < nch)
                def _(b=b, j2=j2):
                    pltpu.async_copy(
                        m_hbm.at[pl.ds(sid * eps + j2 * CHUNK, CHUNK),
                                 pl.ds(col0, D2)], rows[b], sr[b])
            return carry

        lax.fori_loop(0, ngroups, grp, 0)
        plsc.subcore_barrier()
        pltpu.sync_copy(acc.at[pl.ds(sid * rps, rps)],
                        out_hbm.at[pl.ds(sid * rps, rps), pl.ds(col0, D2)])

    return k


def _pad_map(idx, Mp, E, fill, workers):
    K, M = idx.shape
    a = jnp.full((K, Mp), fill, jnp.int32).at[:, :M].set(idx)
    flat = jnp.full((E,), fill, jnp.int32).at[: K * Mp].set(a.reshape(-1))
    return flat.reshape(workers, E // (workers * CHUNK), CHUNK)


def _offset_matmul(g, W, Mp, E):
    K, cin, cout = W.shape

    def body(g_ref, w_ref, o_ref):
        o_ref[...] = jnp.dot(g_ref[...], w_ref[0],
                             preferred_element_type=jnp.float32)

    return pl.pallas_call(
        body,
        grid=(K,),
        in_specs=[
            pl.BlockSpec((Mp, cin), lambda k: (k, 0)),
            pl.BlockSpec((1, cin, cout), lambda k: (k, 0, 0)),
        ],
        out_specs=pl.BlockSpec((Mp, cout), lambda k: (k, 0)),
        out_shape=jax.ShapeDtypeStruct((E, cout), jnp.float32),
    )(g, W)


def _row_block(n):
    if n <= 1024:
        return _rup(n, 8)
    return 1024


def _bn_stats(src, n):
    c = src.shape[1]
    Rb = _row_block(n)
    grid = -(-n // Rb)

    def body(p_ref, st_ref):
        i = pl.program_id(0)
        s = p_ref[...]
        rows = lax.broadcasted_iota(jnp.int32, (Rb, c), 0) + i * Rb
        s = jnp.where(rows < n, s, 0.0)
        blk = jnp.concatenate(
            [jnp.sum(s, axis=0, keepdims=True),
             jnp.sum(s * s, axis=0, keepdims=True),
             jnp.zeros((6, c), jnp.float32)], axis=0)

        @pl.when(i == 0)
        def _():
            st_ref[...] = jnp.zeros_like(st_ref)

        st_ref[...] += blk

    return pl.pallas_call(
        body,
        grid=(grid,),
        in_specs=[pl.BlockSpec((Rb, c), lambda i: (i, 0))],
        out_specs=pl.BlockSpec((8, c), lambda i: (0, 0)),
        out_shape=jax.ShapeDtypeStruct((8, c), jnp.float32),
    )(src)


def _bn_apply(src, st, n, gamma, beta, res, relu):
    c = src.shape[-1]
    Rb = _row_block(n)
    grid = -(-n // Rb)
    inv_n = 1.0 / n

    def body(*refs):
        if res is not None:
            p_ref, st_ref, g_ref, b_ref, r_ref, o_ref = refs
        else:
            p_ref, st_ref, g_ref, b_ref, o_ref = refs
        s = p_ref[...]
        mu = st_ref[0:1] * inv_n
        var = st_ref[1:2] * inv_n - mu * mu
        scale = lax.rsqrt(var + _EPS) * g_ref[...]
        y = (s - mu) * scale + b_ref[...]
        if res is not None:
            y = y + r_ref[...]
        if relu:
            y = jnp.maximum(y, 0.0)
        o_ref[...] = y

    in_specs = [
        pl.BlockSpec((Rb, c), lambda i: (i, 0)),
        pl.BlockSpec((8, c), lambda i: (0, 0)),
        pl.BlockSpec((1, c), lambda i: (0, 0)),
        pl.BlockSpec((1, c), lambda i: (0, 0)),
    ]
    args = [src, st, gamma.reshape(1, c), beta.reshape(1, c)]
    if res is not None:
        in_specs.append(pl.BlockSpec((Rb, c), lambda i: (i, 0)))
        args.append(res)

    return pl.pallas_call(
        body,
        grid=(grid,),
        in_specs=in_specs,
        out_specs=pl.BlockSpec((Rb, c), lambda i: (i, 0)),
        out_shape=jax.ShapeDtypeStruct((n, c), jnp.float32),
    )(*args)


def _matmul_stats(x, W):
    n, cin = x.shape
    cout = W.shape[1]
    Rb = _row_block(n)
    grid = -(-n // Rb)

    def body(x_ref, w_ref, o_ref, st_ref):
        i = pl.program_id(0)
        xw = jnp.dot(x_ref[...], w_ref[...], preferred_element_type=jnp.float32)
        rows = lax.broadcasted_iota(jnp.int32, (Rb, cout), 0) + i * Rb
        xs = jnp.where(rows < n, xw, 0.0)
        blk = jnp.concatenate(
            [jnp.sum(xs, axis=0, keepdims=True),
             jnp.sum(xs * xs, axis=0, keepdims=True),
             jnp.zeros((6, cout), jnp.float32)], axis=0)

        @pl.when(i == 0)
        def _():
            st_ref[...] = jnp.zeros_like(st_ref)

        st_ref[...] += blk
        o_ref[...] = xw

    return pl.pallas_call(
        body,
        grid=(grid,),
        in_specs=[
            pl.BlockSpec((Rb, cin), lambda i: (i, 0)),
            pl.BlockSpec((cin, cout), lambda i: (0, 0)),
        ],
        out_specs=[
            pl.BlockSpec((Rb, cout), lambda i: (i, 0)),
            pl.BlockSpec((8, cout), lambda i: (0, 0)),
        ],
        out_shape=[
            jax.ShapeDtypeStruct((n, cout), jnp.float32),
            jax.ShapeDtypeStruct((8, cout), jnp.float32),
        ],
    )(x, W)


def _final_matmul(x, W, b):
    n, cin = x.shape
    cout = W.shape[1]
    Rb = _row_block(n)
    grid = -(-n // Rb)

    def body(x_ref, w_ref, b_ref, o_ref):
        o_ref[...] = jnp.dot(x_ref[...], w_ref[...],
                             preferred_element_type=jnp.float32) + b_ref[...]

    return pl.pallas_call(
        body,
        grid=(grid,),
        in_specs=[
            pl.BlockSpec((Rb, cin), lambda i: (i, 0)),
            pl.BlockSpec((cin, cout), lambda i: (0, 0)),
            pl.BlockSpec((1, cout), lambda i: (0, 0)),
        ],
        out_specs=pl.BlockSpec((Rb, cout), lambda i: (i, 0)),
        out_shape=jax.ShapeDtypeStruct((n, cout), jnp.float32),
    )(x, W, b.reshape(1, cout))


def _sparse_conv(f, W, iidx, oidx, n_out):
    K, M = iidx.shape
    n_in, cin = f.shape
    cout = W.shape[2]
    if cin % 16:
        pad = 16 - cin % 16
        f = jnp.pad(f, ((0, 0), (0, pad)))
        W = jnp.pad(W, ((0, 0), (0, pad), (0, 0)))
        cin += pad
    Mp = _rup(M, 8)
    E = _rup(K * Mp, EDGE_ALIGN)
    R = _rup(n_out + 1, 128)
    iflat = _pad_map(iidx, Mp, E, 0, NW)
    oflat = _pad_map(oidx, Mp, E, n_out, NS)
    g = _gather_kernel(n_in, cin, E)(f, iflat)
    m = _offset_matmul(g, W, Mp, E)
    zeros = jnp.zeros((R, cout // 2), jnp.float32)
    return _scatter_kernel(E, cout, R)(m, oflat, zeros)


def _sconv_bn(f, W, iidx, oidx, n_out, gamma, beta, res=None, relu=True):
    s = _sparse_conv(f, W, iidx, oidx, n_out)
    st = _bn_stats(s, n_out)
    return _bn_apply(s, st, n_out, gamma, beta, res, relu)


def _matmul_bn(x, W, gamma, beta):
    xw, st = _matmul_stats(x, W)
    return _bn_apply(xw, st, x.shape[0], gamma, beta, None, False)


def _res_block(xf, p, iidx, oidx, n):
    h = _sconv_bn(xf, p["conv1"], iidx, oidx, n, p["bn1g"], p["bn1b"])
    if "down_w" in p:
        res = _matmul_bn(xf, p["down_w"], p["down_g"], p["down_b"])
    else:
        res = xf
    return _sconv_bn(h, p["conv2"], iidx, oidx, n, p["bn2g"], p["bn2b"],
                     res=res, relu=True)


def _res_layer(xf, ps, iidx, oidx, n):
    for p in ps:
        xf = _res_block(xf, p, iidx, oidx, n)
    return xf


def kernel(x, params, km5_in, km5_out, km3_0_in, km3_0_out, km3_1_in,
           km3_1_out, km3_2_in, km3_2_out, km3_3_in, km3_3_out, km3_4_in,
           km3_4_out, kmd01_in, kmd01_out, kmd12_in, kmd12_out, kmd23_in,
           kmd23_out, kmd34_in, kmd34_out):
    P = params
    n0, n1, n2, n3, n4 = 20000, 5000, 1250, 312, 78

    out_p1 = _sconv_bn(x, P["conv0"], km5_in, km5_out, n0,
                       P["bn0g"], P["bn0b"])
    out = _sconv_bn(out_p1, P["conv1"], kmd01_in, kmd01_out, n1,
                    P["bn1g"], P["bn1b"])
    out_b1 = _res_layer(out, P["block1"], km3_1_in, km3_1_out, n1)
    out = _sconv_bn(out_b1, P["conv2"], kmd12_in, kmd12_out, n2,
                    P["bn2g"], P["bn2b"])
    out_b2 = _res_layer(out, P["block2"], km3_2_in, km3_2_out, n2)
    out = _sconv_bn(out_b2, P["conv3"], kmd23_in, kmd23_out, n3,
                    P["bn3g"], P["bn3b"])
    out_b3 = _res_layer(out, P["block3"], km3_3_in, km3_3_out, n3)
    out = _sconv_bn(out_b3, P["conv4"], kmd34_in, kmd34_out, n4,
                    P["bn4g"], P["bn4b"])
    out = _res_layer(out, P["block4"], km3_4_in, km3_4_out, n4)
    out = _sconv_bn(out, P["convtr4"], kmd34_out, kmd34_in, n3,
                    P["bntr4g"], P["bntr4b"])
    out = jnp.concatenate([out, out_b3], axis=1)
    out = _res_layer(out, P["block5"], km3_3_in, km3_3_out, n3)
    out = _sconv_bn(out, P["convtr5"], kmd23_out, kmd23_in, n2,
                    P["bntr5g"], P["bntr5b"])
    out = jnp.concatenate([out, out_b2], axis=1)
    out = _res_layer(out, P["block6"], km3_2_in, km3_2_out, n2)
    out = _sconv_bn(out, P["convtr6"], kmd12_out, kmd12_in, n1,
                    P["bntr6g"], P["bntr6b"])
    out = jnp.concatenate([out, out_b1], axis=1)
    out = _res_layer(out, P["block7"], km3_1_in, km3_1_out, n1)
    out = _sconv_bn(out, P["convtr7"], kmd01_out, kmd01_in, n0,
                    P["bntr7g"], P["bntr7b"])
    out = jnp.concatenate([out, out_p1], axis=1)
    out = _res_layer(out, P["block8"], km3_0_in, km3_0_out, n0)
    return _final_matmul(out, P["final_w"], P["final_b"])


# speedup vs baseline: 1.8271x; 1.0366x over previous
import functools

import jax
import jax.numpy as jnp
from jax import lax
from jax.experimental import pallas as pl
from jax.experimental.pallas import tpu as pltpu
from jax.experimental.pallas import tpu_sc as plsc

NC, NS = 2, 16
NW = NC * NS
CHUNK = 128
EDGE_ALIGN = NW * CHUNK
_EPS = 1e-5


def _rup(x, m):
    return (x + m - 1) // m * m


def _mesh():
    return plsc.VectorSubcoreMesh(core_axis_name="c", subcore_axis_name="s")


_SC_PARAMS = pltpu.CompilerParams(use_tc_tiling_on_sc=False)


def _nbuf(d):
    return max(2, min(8, (400 * 1024) // (CHUNK * d * 4)))


@functools.cache
def _gather_kernel(n_in, D, E):
    epw = E // NW
    nch = epw // CHUNK
    NB = _nbuf(D)
    ngroups = -(-nch // NB)

    @functools.partial(
        pl.kernel,
        out_type=jax.ShapeDtypeStruct((E, D), jnp.float32),
        mesh=_mesh(),
        compiler_params=_SC_PARAMS,
        scratch_types=(
            [pltpu.VMEM((nch, CHUNK), jnp.int32)]
            + [pltpu.VMEM((CHUNK, D), jnp.float32) for _ in range(NB)]
            + [pltpu.SemaphoreType.DMA for _ in range(2 * NB)]
        ),
    )
    def k(f_hbm, idx_hbm, g_hbm, idx_v, *bufs_sems):
        rows = bufs_sems[:NB]
        sg = bufs_sems[NB:2 * NB]
        sw = bufs_sems[2 * NB:]
        wid = lax.axis_index("s") * NC + lax.axis_index("c")
        pltpu.sync_copy(idx_hbm.at[wid], idx_v)
        for b in range(NB):
            if b < nch:
                pltpu.async_copy(f_hbm.at[idx_v.at[b]], rows[b], sg[b])

        def grp(t, carry):
            for b in range(NB):
                j = t * NB + b

                @pl.when(j < nch)
                def _(b=b, j=j):
                    pltpu.make_async_copy(
                        f_hbm.at[pl.ds(0, CHUNK)], rows[b], sg[b]).wait()
                    pltpu.async_copy(
                        rows[b],
                        g_hbm.at[pl.ds(wid * epw + j * CHUNK, CHUNK)], sw[b])
            for b in range(NB):
                j2 = (t + 1) * NB + b

                @pl.when(j2 < nch)
                def _(b=b, j2=j2):
                    pltpu.make_async_copy(
                        rows[b], g_hbm.at[pl.ds(0, CHUNK)], sw[b]).wait()
                    pltpu.async_copy(f_hbm.at[idx_v.at[j2]], rows[b], sg[b])
            return carry

        lax.fori_loop(0, ngroups, grp, 0)
        for b in range(NB):
            if b < nch:
                pltpu.make_async_copy(
                    rows[b], g_hbm.at[pl.ds(0, CHUNK)], sw[b]).wait()

    return k


@functools.cache
def _scatter_kernel(E, D, R):
    D2 = D // 2
    eps = E // NS
    nch = eps // CHUNK
    rps = R // NS
    NB = _nbuf(D2)
    ngroups = -(-nch // NB)

    @functools.partial(
        pl.kernel,
        out_type=jax.ShapeDtypeStruct((R, D), jnp.float32),
        mesh=_mesh(),
        compiler_params=_SC_PARAMS,
        scratch_types=(
            [pltpu.VMEM((nch, CHUNK), jnp.int32),
             pltpu.VMEM_SHARED((R, D2), jnp.float32)]
            + [pltpu.VMEM((CHUNK, D2), jnp.float32) for _ in range(NB)]
            + [pltpu.SemaphoreType.DMA for _ in range(NB)]
        ),
    )
    def k(m_hbm, oidx_hbm, z_hbm, out_hbm, idx_v, acc, *bufs_sems):
        rows = bufs_sems[:NB]
        sr = bufs_sems[NB:]
        cid = lax.axis_index("c")
        sid = lax.axis_index("s")
        col0 = cid * D2
        pltpu.sync_copy(z_hbm.at[pl.ds(sid * rps, rps)],
                        acc.at[pl.ds(sid * rps, rps)])
        plsc.subcore_barrier()
        pltpu.sync_copy(oidx_hbm.at[sid], idx_v)
        for b in range(NB):
            if b < nch:
                pltpu.async_copy(
                    m_hbm.at[pl.ds(sid * eps + b * CHUNK, CHUNK),
                             pl.ds(col0, D2)], rows[b], sr[b])

        def grp(t, carry):
            for b in range(NB):
                j = t * NB + b

                @pl.when(j < nch)
                def _(b=b, j=j):
                    pltpu.make_async_copy(
                        m_hbm.at[pl.ds(0, CHUNK), pl.ds(0, D2)],
                        rows[b], sr[b]).wait()
                    pltpu.sync_copy(rows[b],
                                    acc.at[idx_v.at[j]],
                                    add=True)
            for b in range(NB):
                j2 = (t + 1) * NB + b

                @pl.when(j2 < nch)
                def _(b=b, j2=j2):
                    pltpu.async_copy(
                        m_hbm.at[pl.ds(sid * eps + j2 * CHUNK, CHUNK),
                                 pl.ds(col0, D2)], rows[b], sr[b])
            return carry

        lax.fori_loop(0, ngroups, grp, 0)
        plsc.subcore_barrier()
        pltpu.sync_copy(acc.at[pl.ds(sid * rps, rps)],
                        out_hbm.at[pl.ds(sid * rps, rps), pl.ds(col0, D2)])

    return k


def _pad_map(idx, Mp, E, fill, workers):
    K, M = idx.shape
    a = jnp.full((K, Mp), fill, jnp.int32).at[:, :M].set(idx)
    flat = jnp.full((E,), fill, jnp.int32).at[: K * Mp].set(a.reshape(-1))
    return flat.reshape(workers, E // (workers * CHUNK), CHUNK)


def _offset_matmul(g, W, Mp, E):
    K, cin, cout = W.shape

    def body(g_ref, w_ref, o_ref):
        o_ref[...] = jnp.dot(g_ref[...], w_ref[0],
                             preferred_element_type=jnp.float32)

    return pl.pallas_call(
        body,
        grid=(K,),
        in_specs=[
            pl.BlockSpec((Mp, cin), lambda k: (k, 0)),
            pl.BlockSpec((1, cin, cout), lambda k: (k, 0, 0)),
        ],
        out_specs=pl.BlockSpec((Mp, cout), lambda k: (k, 0)),
        out_shape=jax.ShapeDtypeStruct((E, cout), jnp.float32),
    )(g, W)


def _row_block(n):
    if n <= 1024:
        return _rup(n, 8)
    return 1024


def _bn_fused(src, n, gamma, beta, res=None, relu=True):
    R, c = src.shape
    inv_n = 1.0 / n

    def body(*refs):
        if res is not None:
            p_ref, g_ref, b_ref, r_ref, o_ref = refs
        else:
            p_ref, g_ref, b_ref, o_ref = refs
        s = p_ref[...]
        rows = lax.broadcasted_iota(jnp.int32, (R, c), 0)
        sm = jnp.where(rows < n, s, 0.0)
        mu = jnp.sum(sm, axis=0, keepdims=True) * inv_n
        var = jnp.sum(sm * sm, axis=0, keepdims=True) * inv_n - mu * mu
        scale = lax.rsqrt(var + _EPS) * g_ref[...]
        y = (s - mu) * scale + b_ref[...]
        if res is not None:
            y = y + r_ref[...]
        if relu:
            y = jnp.maximum(y, 0.0)
        o_ref[...] = y

    args = [src, gamma.reshape(1, c), beta.reshape(1, c)]
    if res is not None:
        args.append(res)
    return pl.pallas_call(
        body,
        out_shape=jax.ShapeDtypeStruct((R, c), jnp.float32),
    )(*args)


def _plain_matmul(x, W, R):
    nx, cin = x.shape
    cout = W.shape[1]
    Rb = _row_block(R)
    grid = -(-R // Rb)

    def body(x_ref, w_ref, o_ref):
        o_ref[...] = jnp.dot(x_ref[...], w_ref[...],
                             preferred_element_type=jnp.float32)

    return pl.pallas_call(
        body,
        grid=(grid,),
        in_specs=[
            pl.BlockSpec((Rb, cin), lambda i: (i, 0)),
            pl.BlockSpec((cin, cout), lambda i: (0, 0)),
        ],
        out_specs=pl.BlockSpec((Rb, cout), lambda i: (i, 0)),
        out_shape=jax.ShapeDtypeStruct((R, cout), jnp.float32),
    )(x, W)


def _final_matmul(x, W, b, n):
    cin = x.shape[1]
    cout = W.shape[1]
    Rb = _row_block(n)
    grid = -(-n // Rb)

    def body(x_ref, w_ref, b_ref, o_ref):
        o_ref[...] = jnp.dot(x_ref[...], w_ref[...],
                             preferred_element_type=jnp.float32) + b_ref[...]

    return pl.pallas_call(
        body,
        grid=(grid,),
        in_specs=[
            pl.BlockSpec((Rb, cin), lambda i: (i, 0)),
            pl.BlockSpec((cin, cout), lambda i: (0, 0)),
            pl.BlockSpec((1, cout), lambda i: (0, 0)),
        ],
        out_specs=pl.BlockSpec((Rb, cout), lambda i: (i, 0)),
        out_shape=jax.ShapeDtypeStruct((n, cout), jnp.float32),
    )(x, W, b.reshape(1, cout))


def _sparse_conv(f, W, iidx, oidx, n_out):
    K, M = iidx.shape
    n_in, cin = f.shape
    cout = W.shape[2]
    if cin % 16:
        pad = 16 - cin % 16
        f = jnp.pad(f, ((0, 0), (0, pad)))
        W = jnp.pad(W, ((0, 0), (0, pad), (0, 0)))
        cin += pad
    Mp = _rup(M, 8)
    E = _rup(K * Mp, EDGE_ALIGN)
    R = _rup(n_out + 1, 128)
    iflat = _pad_map(iidx, Mp, E, 0, NW)
    oflat = _pad_map(oidx, Mp, E, n_out, NS)
    g = _gather_kernel(n_in, cin, E)(f, iflat)
    m = _offset_matmul(g, W, Mp, E)
    zeros = jnp.zeros((R, cout // 2), jnp.float32)
    return _scatter_kernel(E, cout, R)(m, oflat, zeros)


def _sconv_bn(f, W, iidx, oidx, n_out, gamma, beta, res=None, relu=True):
    s = _sparse_conv(f, W, iidx, oidx, n_out)
    return _bn_fused(s, n_out, gamma, beta, res, relu)


def _matmul_bn(x, W, gamma, beta, n, R):
    xw = _plain_matmul(x, W, R)
    return _bn_fused(xw, n, gamma, beta, None, relu=False)


def _res_block(xf, p, iidx, oidx, n):
    R = _rup(n + 1, 128)
    h = _sconv_bn(xf, p["conv1"], iidx, oidx, n, p["bn1g"], p["bn1b"])
    if "down_w" in p:
        res = _matmul_bn(xf, p["down_w"], p["down_g"], p["down_b"], n, R)
    else:
        res = xf
    return _sconv_bn(h, p["conv2"], iidx, oidx, n, p["bn2g"], p["bn2b"],
                     res=res, relu=True)


def _res_layer(xf, ps, iidx, oidx, n):
    for p in ps:
        xf = _res_block(xf, p, iidx, oidx, n)
    return xf


def kernel(x, params, km5_in, km5_out, km3_0_in, km3_0_out, km3_1_in,
           km3_1_out, km3_2_in, km3_2_out, km3_3_in, km3_3_out, km3_4_in,
           km3_4_out, kmd01_in, kmd01_out, kmd12_in, kmd12_out, kmd23_in,
           kmd23_out, kmd34_in, kmd34_out):
    P = params
    n0, n1, n2, n3, n4 = 20000, 5000, 1250, 312, 78

    out_p1 = _sconv_bn(x, P["conv0"], km5_in, km5_out, n0,
                       P["bn0g"], P["bn0b"])
    out = _sconv_bn(out_p1, P["conv1"], kmd01_in, kmd01_out, n1,
                    P["bn1g"], P["bn1b"])
    out_b1 = _res_layer(out, P["block1"], km3_1_in, km3_1_out, n1)
    out = _sconv_bn(out_b1, P["conv2"], kmd12_in, kmd12_out, n2,
                    P["bn2g"], P["bn2b"])
    out_b2 = _res_layer(out, P["block2"], km3_2_in, km3_2_out, n2)
    out = _sconv_bn(out_b2, P["conv3"], kmd23_in, kmd23_out, n3,
                    P["bn3g"], P["bn3b"])
    out_b3 = _res_layer(out, P["block3"], km3_3_in, km3_3_out, n3)
    out = _sconv_bn(out_b3, P["conv4"], kmd34_in, kmd34_out, n4,
                    P["bn4g"], P["bn4b"])
    out = _res_layer(out, P["block4"], km3_4_in, km3_4_out, n4)
    out = _sconv_bn(out, P["convtr4"], kmd34_out, kmd34_in, n3,
                    P["bntr4g"], P["bntr4b"])
    out = jnp.concatenate([out[:n3], out_b3[:n3]], axis=1)
    out = _res_layer(out, P["block5"], km3_3_in, km3_3_out, n3)
    out = _sconv_bn(out, P["convtr5"], kmd23_out, kmd23_in, n2,
                    P["bntr5g"], P["bntr5b"])
    out = jnp.concatenate([out[:n2], out_b2[:n2]], axis=1)
    out = _res_layer(out, P["block6"], km3_2_in, km3_2_out, n2)
    out = _sconv_bn(out, P["convtr6"], kmd12_out, kmd12_in, n1,
                    P["bntr6g"], P["bntr6b"])
    out = jnp.concatenate([out[:n1], out_b1[:n1]], axis=1)
    out = _res_layer(out, P["block7"], km3_1_in, km3_1_out, n1)
    out = _sconv_bn(out, P["convtr7"], kmd01_out, kmd01_in, n0,
                    P["bntr7g"], P["bntr7b"])
    out = jnp.concatenate([out[:n0], out_p1[:n0]], axis=1)
    out = _res_layer(out, P["block8"], km3_0_in, km3_0_out, n0)
    return _final_matmul(out, P["final_w"], P["final_b"], n0)


# speedup vs baseline: 2.9812x; 1.6316x over previous
import functools

import jax
import jax.numpy as jnp
from jax import lax
from jax.experimental import pallas as pl
from jax.experimental.pallas import tpu as pltpu
from jax.experimental.pallas import tpu_sc as plsc

NC, NS = 2, 16
NW = NC * NS
CHUNK = 128
EDGE_ALIGN = NW * CHUNK
_EPS = 1e-5


def _rup(x, m):
    return (x + m - 1) // m * m


def _mesh():
    return plsc.VectorSubcoreMesh(core_axis_name="c", subcore_axis_name="s")


_SC_PARAMS = pltpu.CompilerParams(use_tc_tiling_on_sc=False)


def _nbuf(d):
    return max(2, min(8, (400 * 1024) // (CHUNK * d * 4)))


@functools.cache
def _gather_kernel(n_in, D, E):
    epw = E // NW
    nch = epw // CHUNK
    NB = _nbuf(D)
    ngroups = -(-nch // NB)

    @functools.partial(
        pl.kernel,
        out_type=jax.ShapeDtypeStruct((E, D), jnp.float32),
        mesh=_mesh(),
        compiler_params=_SC_PARAMS,
        scratch_types=(
            [pltpu.VMEM((nch, CHUNK), jnp.int32)]
            + [pltpu.VMEM((CHUNK, D), jnp.float32) for _ in range(NB)]
            + [pltpu.SemaphoreType.DMA for _ in range(2 * NB)]
        ),
    )
    def k(f_hbm, idx_hbm, g_hbm, idx_v, *bufs_sems):
        rows = bufs_sems[:NB]
        sg = bufs_sems[NB:2 * NB]
        sw = bufs_sems[2 * NB:]
        wid = lax.axis_index("s") * NC + lax.axis_index("c")
        pltpu.sync_copy(idx_hbm.at[wid], idx_v)
        for b in range(NB):
            if b < nch:
                pltpu.async_copy(f_hbm.at[idx_v.at[b]], rows[b], sg[b])

        def grp(t, carry):
            for b in range(NB):
                j = t * NB + b

                @pl.when(j < nch)
                def _(b=b, j=j):
                    pltpu.make_async_copy(
                        f_hbm.at[pl.ds(0, CHUNK)], rows[b], sg[b]).wait()
                    pltpu.async_copy(
                        rows[b],
                        g_hbm.at[pl.ds(wid * epw + j * CHUNK, CHUNK)], sw[b])
            for b in range(NB):
                j2 = (t + 1) * NB + b

                @pl.when(j2 < nch)
                def _(b=b, j2=j2):
                    pltpu.make_async_copy(
                        rows[b], g_hbm.at[pl.ds(0, CHUNK)], sw[b]).wait()
                    pltpu.async_copy(f_hbm.at[idx_v.at[j2]], rows[b], sg[b])
            return carry

        lax.fori_loop(0, ngroups, grp, 0)
        for b in range(NB):
            if b < nch:
                pltpu.make_async_copy(
                    rows[b], g_hbm.at[pl.ds(0, CHUNK)], sw[b]).wait()

    return k


@functools.cache
def _scatter_kernel(E, D, R):
    D2 = D // 2
    eps = E // NS
    nch = eps // CHUNK
    rps = R // NS
    NB = _nbuf(D2)
    ngroups = -(-nch // NB)

    @functools.partial(
        pl.kernel,
        out_type=jax.ShapeDtypeStruct((R, D), jnp.float32),
        mesh=_mesh(),
        compiler_params=_SC_PARAMS,
        scratch_types=(
            [pltpu.VMEM((nch, CHUNK), jnp.int32),
             pltpu.VMEM_SHARED((R, D2), jnp.float32)]
            + [pltpu.VMEM((CHUNK, D2), jnp.float32) for _ in range(NB)]
            + [pltpu.SemaphoreType.DMA for _ in range(NB)]
        ),
    )
    def k(m_hbm, oidx_hbm, z_hbm, out_hbm, idx_v, acc, *bufs_sems):
        rows = bufs_sems[:NB]
        sr = bufs_sems[NB:]
        cid = lax.axis_index("c")
        sid = lax.axis_index("s")
        col0 = cid * D2
        pltpu.sync_copy(z_hbm.at[pl.ds(sid * rps, rps)],
                        acc.at[pl.ds(sid * rps, rps)])
        plsc.subcore_barrier()
        pltpu.sync_copy(oidx_hbm.at[sid], idx_v)
        for b in range(NB):
            if b < nch:
                pltpu.async_copy(
                    m_hbm.at[pl.ds(sid * eps + b * CHUNK, CHUNK),
                             pl.ds(col0, D2)], rows[b], sr[b])

        def grp(t, carry):
            for b in range(NB):
                j = t * NB + b

                @pl.when(j < nch)
                def _(b=b, j=j):
                    pltpu.make_async_copy(
                        m_hbm.at[pl.ds(0, CHUNK), pl.ds(0, D2)],
                        rows[b], sr[b]).wait()
                    pltpu.sync_copy(rows[b],
                                    acc.at[idx_v.at[j]],
                                    add=True)
            for b in range(NB):
                j2 = (t + 1) * NB + b

                @pl.when(j2 < nch)
                def _(b=b, j2=j2):
                    pltpu.async_copy(
                        m_hbm.at[pl.ds(sid * eps + j2 * CHUNK, CHUNK),
                                 pl.ds(col0, D2)], rows[b], sr[b])
            return carry

        lax.fori_loop(0, ngroups, grp, 0)
        plsc.subcore_barrier()
        pltpu.sync_copy(acc.at[pl.ds(sid * rps, rps)],
                        out_hbm.at[pl.ds(sid * rps, rps), pl.ds(col0, D2)])

    return k


def _pad_map(idx, Mp, E, fill, workers):
    K, M = idx.shape
    a = jnp.full((K, Mp), fill, jnp.int32).at[:, :M].set(idx)
    flat = jnp.full((E,), fill, jnp.int32).at[: K * Mp].set(a.reshape(-1))
    return flat.reshape(workers, E // (workers * CHUNK), CHUNK)


def _offset_matmul(g, W, Mp, E):
    K, cin, cout = W.shape

    def body(g_ref, w_ref, o_ref):
        o_ref[...] = jnp.dot(g_ref[...], w_ref[0],
                             preferred_element_type=jnp.float32)

    return pl.pallas_call(
        body,
        grid=(K,),
        in_specs=[
            pl.BlockSpec((Mp, cin), lambda k: (k, 0)),
            pl.BlockSpec((1, cin, cout), lambda k: (k, 0, 0)),
        ],
        out_specs=pl.BlockSpec((Mp, cout), lambda k: (k, 0)),
        out_shape=jax.ShapeDtypeStruct((E, cout), jnp.float32),
    )(g, W)


def _row_block(n):
    if n <= 1024:
        return _rup(n, 8)
    return 1024


def _bn_fused(src, n, gamma, beta, res=None, relu=True):
    R, c = src.shape
    inv_n = 1.0 / n

    def body(*refs):
        if res is not None:
            p_ref, g_ref, b_ref, r_ref, o_ref = refs
        else:
            p_ref, g_ref, b_ref, o_ref = refs
        s = p_ref[...]
        rows = lax.broadcasted_iota(jnp.int32, (R, c), 0)
        sm = jnp.where(rows < n, s, 0.0)
        mu = jnp.sum(sm, axis=0, keepdims=True) * inv_n
        var = jnp.sum(sm * sm, axis=0, keepdims=True) * inv_n - mu * mu
        scale = lax.rsqrt(var + _EPS) * g_ref[...]
        y = (s - mu) * scale + b_ref[...]
        if res is not None:
            y = y + r_ref[...]
        if relu:
            y = jnp.maximum(y, 0.0)
        o_ref[...] = y

    args = [src, gamma.reshape(1, c), beta.reshape(1, c)]
    if res is not None:
        args.append(res)
    return pl.pallas_call(
        body,
        out_shape=jax.ShapeDtypeStruct((R, c), jnp.float32),
    )(*args)


def _plain_matmul(x, W, R):
    nx, cin = x.shape
    cout = W.shape[1]
    Rb = _row_block(R)
    grid = -(-R // Rb)

    def body(x_ref, w_ref, o_ref):
        o_ref[...] = jnp.dot(x_ref[...], w_ref[...],
                             preferred_element_type=jnp.float32)

    return pl.pallas_call(
        body,
        grid=(grid,),
        in_specs=[
            pl.BlockSpec((Rb, cin), lambda i: (i, 0)),
            pl.BlockSpec((cin, cout), lambda i: (0, 0)),
        ],
        out_specs=pl.BlockSpec((Rb, cout), lambda i: (i, 0)),
        out_shape=jax.ShapeDtypeStruct((R, cout), jnp.float32),
    )(x, W)


def _final_matmul(x, W, b, n):
    cin = x.shape[1]
    cout = W.shape[1]
    Rb = _row_block(n)
    grid = -(-n // Rb)

    def body(x_ref, w_ref, b_ref, o_ref):
        o_ref[...] = jnp.dot(x_ref[...], w_ref[...],
                             preferred_element_type=jnp.float32) + b_ref[...]

    return pl.pallas_call(
        body,
        grid=(grid,),
        in_specs=[
            pl.BlockSpec((Rb, cin), lambda i: (i, 0)),
            pl.BlockSpec((cin, cout), lambda i: (0, 0)),
            pl.BlockSpec((1, cout), lambda i: (0, 0)),
        ],
        out_specs=pl.BlockSpec((Rb, cout), lambda i: (i, 0)),
        out_shape=jax.ShapeDtypeStruct((n, cout), jnp.float32),
    )(x, W, b.reshape(1, cout))


def _dense_conv_bn(f, W, iidx, oidx, n_out, gamma, beta, res=None, relu=True):
    K, M = iidx.shape
    n_in, cin = f.shape
    cout = W.shape[2]
    inv_n = 1.0 / n_out
    ii3 = iidx.reshape(K, 1, M)
    oi3 = oidx.reshape(K, 1, M)

    def body(*refs):
        if res is not None:
            f_ref, ii_ref, oi_ref, w_ref, g_ref, b_ref, r_ref, o_ref = refs
        else:
            f_ref, ii_ref, oi_ref, w_ref, g_ref, b_ref, o_ref = refs
        k = pl.program_id(0)
        ii = ii_ref[0, 0]
        oi = oi_ref[0, 0]
        ohi = (lax.broadcasted_iota(jnp.int32, (M, n_in), 1)
               == ii[:, None]).astype(jnp.float32)
        g = jnp.dot(ohi, f_ref[...], preferred_element_type=jnp.float32)
        mk = jnp.dot(g, w_ref[0], preferred_element_type=jnp.float32)
        oho = (lax.broadcasted_iota(jnp.int32, (n_out, M), 0)
               == oi[None, :]).astype(jnp.float32)
        contrib = jnp.dot(oho, mk, preferred_element_type=jnp.float32)

        @pl.when(k == 0)
        def _():
            o_ref[...] = contrib

        @pl.when((k > 0) & (k < K - 1))
        def _():
            o_ref[...] += contrib

        @pl.when(k == K - 1)
        def _():
            s = o_ref[...] + contrib
            mu = jnp.sum(s, axis=0, keepdims=True) * inv_n
            var = jnp.sum(s * s, axis=0, keepdims=True) * inv_n - mu * mu
            y = (s - mu) * (lax.rsqrt(var + _EPS) * g_ref[...]) + b_ref[...]
            if res is not None:
                y = y + r_ref[...]
            if relu:
                y = jnp.maximum(y, 0.0)
            o_ref[...] = y

    c = cout
    in_specs = [
        pl.BlockSpec((n_in, cin), lambda k: (0, 0)),
        pl.BlockSpec((1, 1, M), lambda k: (k, 0, 0)),
        pl.BlockSpec((1, 1, M), lambda k: (k, 0, 0)),
        pl.BlockSpec((1, cin, cout), lambda k: (k, 0, 0)),
        pl.BlockSpec((1, c), lambda k: (0, 0)),
        pl.BlockSpec((1, c), lambda k: (0, 0)),
    ]
    args = [f, ii3, oi3, W, gamma.reshape(1, c), beta.reshape(1, c)]
    if res is not None:
        in_specs.append(pl.BlockSpec((n_out, c), lambda k: (0, 0)))
        args.append(res)

    return pl.pallas_call(
        body,
        grid=(K,),
        in_specs=in_specs,
        out_specs=pl.BlockSpec((n_out, c), lambda k: (0, 0)),
        out_shape=jax.ShapeDtypeStruct((n_out, c), jnp.float32),
    )(*args)


def _sparse_conv(f, W, iidx, oidx, n_out):
    K, M = iidx.shape
    n_in, cin = f.shape
    cout = W.shape[2]
    if cin % 16:
        pad = 16 - cin % 16
        f = jnp.pad(f, ((0, 0), (0, pad)))
        W = jnp.pad(W, ((0, 0), (0, pad), (0, 0)))
        cin += pad
    Mp = _rup(M, 8)
    E = _rup(K * Mp, EDGE_ALIGN)
    R = _rup(n_out + 1, 128)
    iflat = _pad_map(iidx, Mp, E, 0, NW)
    oflat = _pad_map(oidx, Mp, E, n_out, NS)
    g = _gather_kernel(n_in, cin, E)(f, iflat)
    m = _offset_matmul(g, W, Mp, E)
    zeros = jnp.zeros((R, cout // 2), jnp.float32)
    return _scatter_kernel(E, cout, R)(m, oflat, zeros)


def _sconv_bn(f, W, iidx, oidx, n_out, gamma, beta, res=None, relu=True):
    if n_out <= 1300 and f.shape[0] <= 1300:
        if res is not None and res.shape[0] != n_out:
            res = res[:n_out]
        return _dense_conv_bn(f, W, iidx, oidx, n_out, gamma, beta, res, relu)
    s = _sparse_conv(f, W, iidx, oidx, n_out)
    return _bn_fused(s, n_out, gamma, beta, res, relu)


def _matmul_bn(x, W, gamma, beta, n, R):
    xw = _plain_matmul(x, W, R)
    return _bn_fused(xw, n, gamma, beta, None, relu=False)


def _res_block(xf, p, iidx, oidx, n):
    dense = n <= 1300 and xf.shape[0] <= 1300
    R = n if dense else _rup(n + 1, 128)
    h = _sconv_bn(xf, p["conv1"], iidx, oidx, n, p["bn1g"], p["bn1b"])
    if "down_w" in p:
        res = _matmul_bn(xf, p["down_w"], p["down_g"], p["down_b"], n, R)
    else:
        res = xf
    return _sconv_bn(h, p["conv2"], iidx, oidx, n, p["bn2g"], p["bn2b"],
                     res=res, relu=True)


def _res_layer(xf, ps, iidx, oidx, n):
    for p in ps:
        xf = _res_block(xf, p, iidx, oidx, n)
    return xf


def kernel(x, params, km5_in, km5_out, km3_0_in, km3_0_out, km3_1_in,
           km3_1_out, km3_2_in, km3_2_out, km3_3_in, km3_3_out, km3_4_in,
           km3_4_out, kmd01_in, kmd01_out, kmd12_in, kmd12_out, kmd23_in,
           kmd23_out, kmd34_in, kmd34_out):
    P = params
    n0, n1, n2, n3, n4 = 20000, 5000, 1250, 312, 78

    out_p1 = _sconv_bn(x, P["conv0"], km5_in, km5_out, n0,
                       P["bn0g"], P["bn0b"])
    out = _sconv_bn(out_p1, P["conv1"], kmd01_in, kmd01_out, n1,
                    P["bn1g"], P["bn1b"])
    out_b1 = _res_layer(out, P["block1"], km3_1_in, km3_1_out, n1)
    out = _sconv_bn(out_b1, P["conv2"], kmd12_in, kmd12_out, n2,
                    P["bn2g"], P["bn2b"])[:n2]
    out_b2 = _res_layer(out, P["block2"], km3_2_in, km3_2_out, n2)
    out = _sconv_bn(out_b2, P["conv3"], kmd23_in, kmd23_out, n3,
                    P["bn3g"], P["bn3b"])
    out_b3 = _res_layer(out, P["block3"], km3_3_in, km3_3_out, n3)
    out = _sconv_bn(out_b3, P["conv4"], kmd34_in, kmd34_out, n4,
                    P["bn4g"], P["bn4b"])
    out = _res_layer(out, P["block4"], km3_4_in, km3_4_out, n4)
    out = _sconv_bn(out, P["convtr4"], kmd34_out, kmd34_in, n3,
                    P["bntr4g"], P["bntr4b"])
    out = jnp.concatenate([out[:n3], out_b3[:n3]], axis=1)
    out = _res_layer(out, P["block5"], km3_3_in, km3_3_out, n3)
    out = _sconv_bn(out, P["convtr5"], kmd23_out, kmd23_in, n2,
                    P["bntr5g"], P["bntr5b"])
    out = jnp.concatenate([out[:n2], out_b2[:n2]], axis=1)
    out = _res_layer(out, P["block6"], km3_2_in, km3_2_out, n2)
    out = _sconv_bn(out, P["convtr6"], kmd12_out, kmd12_in, n1,
                    P["bntr6g"], P["bntr6b"])
    out = jnp.concatenate([out[:n1], out_b1[:n1]], axis=1)
    out = _res_layer(out, P["block7"], km3_1_in, km3_1_out, n1)
    out = _sconv_bn(out, P["convtr7"], kmd01_out, kmd01_in, n0,
                    P["bntr7g"], P["bntr7b"])
    out = jnp.concatenate([out[:n0], out_p1[:n0]], axis=1)
    out = _res_layer(out, P["block8"], km3_0_in, km3_0_out, n0)
    return _final_matmul(out, P["final_w"], P["final_b"], n0)
